# Initial kernel scaffold; baseline (speedup 1.0000x reference)
import jax
import jax.numpy as jnp
from jax.experimental import pallas as pl


def kernel(xyz, g, tp, sa1_b0_l0_W, sa1_b0_l0_b, sa1_b0_l0_g, sa1_b0_l0_beta, sa1_b0_l1_W, sa1_b0_l1_b, sa1_b0_l1_g, sa1_b0_l1_beta, sa1_b1_l0_W, sa1_b1_l0_b, sa1_b1_l0_g, sa1_b1_l0_beta, sa1_b1_l1_W, sa1_b1_l1_b, sa1_b1_l1_g, sa1_b1_l1_beta, sa1_b2_l0_W, sa1_b2_l0_b, sa1_b2_l0_g, sa1_b2_l0_beta, sa1_b2_l1_W, sa1_b2_l1_b, sa1_b2_l1_g, sa1_b2_l1_beta, fp_l0_W, fp_l0_b, fp_l0_g, fp_l0_beta, fp_l1_W, fp_l1_b, fp_l1_g, fp_l1_beta, conv_W, conv_b):
    raise NotImplementedError("write your pallas kernel here")


# speedup vs baseline: 11.5757x; 11.5757x over previous
import jax
import jax.numpy as jnp
from jax.experimental import pallas as pl

B = 8
N = 32768
S = 36
K = 36
NT = 4096
G_CH = 64
F32 = jnp.float32


def _fps_kernel(xyz_ref, out_ref):
    xyz = xyz_ref[0]
    lane = jax.lax.broadcasted_iota(jnp.int32, (1, N), 1)
    lane_s = jax.lax.broadcasted_iota(jnp.int32, (1, S), 1)
    dist = jnp.full((1, N), 1e10, dtype=F32)
    f = jnp.int32(0)
    acc = jnp.zeros((1, S), dtype=jnp.int32)
    for i in range(S):
        acc = jnp.where(lane_s == i, f, acc)
        oh = (lane == f).astype(F32)
        cen = jax.lax.dot_general(xyz, oh, (((1,), (1,)), ((), ())),
                                  preferred_element_type=F32)
        d = ((xyz[0:1] - cen[0:1]) ** 2
             + (xyz[1:2] - cen[1:2]) ** 2
             + (xyz[2:3] - cen[2:3]) ** 2)
        dist = jnp.minimum(dist, d)
        m = jnp.max(dist)
        f = jnp.min(jnp.where(dist == m, lane, N)).astype(jnp.int32)
    out_ref[0] = acc


def _ballq_kernel(r2_ref, xyz_ref, c_ref, out_ref):
    r2 = r2_ref[0, 0, 0]
    xyz = xyz_ref[0]
    c = c_ref[0]
    xx = jnp.sum(xyz * xyz, axis=0, keepdims=True)
    cc = jnp.sum(c * c, axis=1, keepdims=True)
    cx = jax.lax.dot_general(c, xyz, (((1,), (0,)), ((), ())),
                             preferred_element_type=F32)
    d = cc + xx - 2.0 * cx
    lane = jax.lax.broadcasted_iota(jnp.int32, (S, N), 1)
    masked = jnp.where(d <= r2, lane, N)
    cols = []
    for _ in range(K):
        cur = jnp.min(masked, axis=1, keepdims=True)
        cols.append(cur)
        masked = jnp.where(masked == cur, N, masked)
    gi = jnp.concatenate(cols, axis=1)
    gi = jnp.where(gi == N, gi[:, 0:1], gi)
    out_ref[...] = gi[None, None]


def _bn_relu(z, g, t):
    m = jnp.mean(z, axis=0, keepdims=True)
    v = jnp.mean((z - m) ** 2, axis=0, keepdims=True)
    return jnp.maximum((z - m) / jnp.sqrt(v + 1e-5) * g + t, 0.0)


def _sa_kernel(x0_ref, w00_ref, b00_ref, g00_ref, t00_ref,
               w01_ref, b01_ref, g01_ref, t01_ref,
               x1_ref, w10_ref, b10_ref, g10_ref, t10_ref,
               w11_ref, b11_ref, g11_ref, t11_ref,
               x2_ref, w20_ref, b20_ref, g20_ref, t20_ref,
               w21_ref, b21_ref, g21_ref, t21_ref,
               out_ref):
    def branch(x_ref, wa_ref, ba_ref, ga_ref, ta_ref,
               wb_ref, bb_ref, gb_ref, tb_ref):
        x = x_ref[...]
        z = jax.lax.dot_general(x, wa_ref[...], (((1,), (1,)), ((), ())),
                                preferred_element_type=F32) + ba_ref[...]
        z = _bn_relu(z, ga_ref[...], ta_ref[...])
        z = jax.lax.dot_general(z, wb_ref[...], (((1,), (1,)), ((), ())),
                                preferred_element_type=F32) + bb_ref[...]
        z = _bn_relu(z, gb_ref[...], tb_ref[...])
        m = z[0:B * S]
        for k in range(1, K):
            m = jnp.maximum(m, z[k * B * S:(k + 1) * B * S])
        return m

    o0 = branch(x0_ref, w00_ref, b00_ref, g00_ref, t00_ref,
                w01_ref, b01_ref, g01_ref, t01_ref)
    o1 = branch(x1_ref, w10_ref, b10_ref, g10_ref, t10_ref,
                w11_ref, b11_ref, g11_ref, t11_ref)
    o2 = branch(x2_ref, w20_ref, b20_ref, g20_ref, t20_ref,
                w21_ref, b21_ref, g21_ref, t21_ref)
    out_ref[...] = jnp.concatenate([o0, o1, o2], axis=1)


def _fp1_kernel(tp_ref, c_ref, p2_ref, g_ref, w0_ref, b0_ref,
                z_ref, s_ref, q_ref):
    t = tp_ref[0]
    c = c_ref[0]
    tt = jnp.sum(t * t, axis=1, keepdims=True)
    cc = jnp.sum(c * c, axis=1)[None, :]
    tc = jax.lax.dot_general(t, c, (((1,), (1,)), ((), ())),
                             preferred_element_type=F32)
    d = tt + cc - 2.0 * tc
    lane = jax.lax.broadcasted_iota(jnp.int32, (NT, S), 1)
    work = d
    recs, ohs = [], []
    for _ in range(3):
        m = jnp.min(work, axis=1, keepdims=True)
        idx = jnp.min(jnp.where(work == m, lane, S), axis=1, keepdims=True)
        oh = (lane == idx).astype(F32)
        recs.append(1.0 / (m + 1e-8))
        ohs.append(oh)
        work = jnp.where(lane == idx, jnp.float32(1e30), work)
    rsum = recs[0] + recs[1] + recs[2]
    wmat = ((recs[0] / rsum) * ohs[0] + (recs[1] / rsum) * ohs[1]
            + (recs[2] / rsum) * ohs[2])
    interp = jax.lax.dot_general(wmat, p2_ref[0], (((1,), (0,)), ((), ())),
                                 preferred_element_type=F32)
    w0 = w0_ref[...]
    z = jax.lax.dot_general(interp, w0[:, G_CH:], (((1,), (1,)), ((), ())),
                            preferred_element_type=F32)
    gpart = jax.lax.dot_general(w0[:, :G_CH], g_ref[0],
                                (((1,), (0,)), ((), ())),
                                preferred_element_type=F32)
    z = z + jnp.transpose(gpart) + b0_ref[...]
    z_ref[0] = z
    s_ref[0] = jnp.sum(z, axis=0, keepdims=True)
    q_ref[0] = jnp.sum(z * z, axis=0, keepdims=True)


def _fp2_kernel(z_ref, m_ref, v_ref, g_ref, t_ref, w_ref, b_ref,
                z1_ref, s_ref, q_ref):
    z = z_ref[0]
    h = jnp.maximum((z - m_ref[...]) / jnp.sqrt(v_ref[...] + 1e-5)
                    * g_ref[...] + t_ref[...], 0.0)
    z1 = jax.lax.dot_general(h, w_ref[...], (((1,), (1,)), ((), ())),
                             preferred_element_type=F32) + b_ref[...]
    z1_ref[0] = z1
    s_ref[0] = jnp.sum(z1, axis=0, keepdims=True)
    q_ref[0] = jnp.sum(z1 * z1, axis=0, keepdims=True)


def _fp3_kernel(z_ref, m_ref, v_ref, g_ref, t_ref, w_ref, b_ref, out_ref):
    z = z_ref[0]
    h = jnp.maximum((z - m_ref[...]) / jnp.sqrt(v_ref[...] + 1e-5)
                    * g_ref[...] + t_ref[...], 0.0)
    out = jax.lax.dot_general(w_ref[...], h, (((1,), (1,)), ((), ())),
                              preferred_element_type=F32)
    out_ref[0] = out + b_ref[...]


def kernel(xyz, g, tp, sa1_b0_l0_W, sa1_b0_l0_b, sa1_b0_l0_g, sa1_b0_l0_beta, sa1_b0_l1_W, sa1_b0_l1_b, sa1_b0_l1_g, sa1_b0_l1_beta, sa1_b1_l0_W, sa1_b1_l0_b, sa1_b1_l0_g, sa1_b1_l0_beta, sa1_b1_l1_W, sa1_b1_l1_b, sa1_b1_l1_g, sa1_b1_l1_beta, sa1_b2_l0_W, sa1_b2_l0_b, sa1_b2_l0_g, sa1_b2_l0_beta, sa1_b2_l1_W, sa1_b2_l1_b, sa1_b2_l1_g, sa1_b2_l1_beta, fp_l0_W, fp_l0_b, fp_l0_g, fp_l0_beta, fp_l1_W, fp_l1_b, fp_l1_g, fp_l1_beta, conv_W, conv_b):
    xyz3 = xyz[:, :3, :]

    fps_idx = pl.pallas_call(
        _fps_kernel,
        grid=(B,),
        in_specs=[pl.BlockSpec((1, 3, N), lambda b: (b, 0, 0))],
        out_specs=pl.BlockSpec((1, 1, S), lambda b: (b, 0, 0)),
        out_shape=jax.ShapeDtypeStruct((B, 1, S), jnp.int32),
    )(xyz3).reshape(B, S)

    xyzT = jnp.transpose(xyz3, (0, 2, 1))
    barange = jnp.arange(B)[:, None]
    new_xyz = xyzT[barange, fps_idx]

    r2 = jnp.array([0.1 ** 2, 0.2 ** 2, 0.4 ** 2], dtype=F32).reshape(3, 1, 1)
    gi = pl.pallas_call(
        _ballq_kernel,
        grid=(B, 3),
        in_specs=[
            pl.BlockSpec((1, 1, 1), lambda b, r: (r, 0, 0)),
            pl.BlockSpec((1, 3, N), lambda b, r: (b, 0, 0)),
            pl.BlockSpec((1, S, 3), lambda b, r: (b, 0, 0)),
        ],
        out_specs=pl.BlockSpec((1, 1, S, K), lambda b, r: (b, r, 0, 0)),
        out_shape=jax.ShapeDtypeStruct((B, 3, S, K), jnp.int32),
    )(r2, xyz3, new_xyz)

    ptsT = jnp.transpose(xyz, (0, 2, 1))
    xs = []
    for r in range(3):
        gir = gi[:, r]
        gpts = ptsT[barange[:, :, None], gir]
        gxyz = xyzT[barange[:, :, None], gir] - new_xyz[:, :, None, :]
        gp = jnp.concatenate([gpts, gxyz], axis=-1)
        gp = jnp.transpose(gp, (2, 0, 1, 3)).reshape(K * B * S, 9)
        xs.append(gp)

    def row(v):
        return v.reshape(1, -1)

    l1 = pl.pallas_call(
        _sa_kernel,
        out_shape=jax.ShapeDtypeStruct((B * S, 512), F32),
    )(xs[0], sa1_b0_l0_W, row(sa1_b0_l0_b), row(sa1_b0_l0_g), row(sa1_b0_l0_beta),
      sa1_b0_l1_W, row(sa1_b0_l1_b), row(sa1_b0_l1_g), row(sa1_b0_l1_beta),
      xs[1], sa1_b1_l0_W, row(sa1_b1_l0_b), row(sa1_b1_l0_g), row(sa1_b1_l0_beta),
      sa1_b1_l1_W, row(sa1_b1_l1_b), row(sa1_b1_l1_g), row(sa1_b1_l1_beta),
      xs[2], sa1_b2_l0_W, row(sa1_b2_l0_b), row(sa1_b2_l0_g), row(sa1_b2_l0_beta),
      sa1_b2_l1_W, row(sa1_b2_l1_b), row(sa1_b2_l1_g), row(sa1_b2_l1_beta))
    p2 = l1.reshape(B, S, 512)

    tpT = jnp.transpose(tp, (0, 2, 1))

    z0, s0, q0 = pl.pallas_call(
        _fp1_kernel,
        grid=(B,),
        in_specs=[
            pl.BlockSpec((1, NT, 3), lambda b: (b, 0, 0)),
            pl.BlockSpec((1, S, 3), lambda b: (b, 0, 0)),
            pl.BlockSpec((1, S, 512), lambda b: (b, 0, 0)),
            pl.BlockSpec((1, G_CH, 1), lambda b: (b, 0, 0)),
            pl.BlockSpec((512, 512 + G_CH), lambda b: (0, 0)),
            pl.BlockSpec((1, 512), lambda b: (0, 0)),
        ],
        out_specs=[
            pl.BlockSpec((1, NT, 512), lambda b: (b, 0, 0)),
            pl.BlockSpec((1, 1, 512), lambda b: (b, 0, 0)),
            pl.BlockSpec((1, 1, 512), lambda b: (b, 0, 0)),
        ],
        out_shape=[
            jax.ShapeDtypeStruct((B, NT, 512), F32),
            jax.ShapeDtypeStruct((B, 1, 512), F32),
            jax.ShapeDtypeStruct((B, 1, 512), F32),
        ],
    )(tpT, new_xyz, p2, g, fp_l0_W, row(fp_l0_b))

    cnt = float(B * NT)
    mean0 = jnp.sum(s0, axis=0) / cnt
    var0 = jnp.sum(q0, axis=0) / cnt - mean0 ** 2

    z1, s1, q1 = pl.pallas_call(
        _fp2_kernel,
        grid=(B,),
        in_specs=[
            pl.BlockSpec((1, NT, 512), lambda b: (b, 0, 0)),
            pl.BlockSpec((1, 512), lambda b: (0, 0)),
            pl.BlockSpec((1, 512), lambda b: (0, 0)),
            pl.BlockSpec((1, 512), lambda b: (0, 0)),
            pl.BlockSpec((1, 512), lambda b: (0, 0)),
            pl.BlockSpec((256, 512), lambda b: (0, 0)),
            pl.BlockSpec((1, 256), lambda b: (0, 0)),
        ],
        out_specs=[
            pl.BlockSpec((1, NT, 256), lambda b: (b, 0, 0)),
            pl.BlockSpec((1, 1, 256), lambda b: (b, 0, 0)),
            pl.BlockSpec((1, 1, 256), lambda b: (b, 0, 0)),
        ],
        out_shape=[
            jax.ShapeDtypeStruct((B, NT, 256), F32),
            jax.ShapeDtypeStruct((B, 1, 256), F32),
            jax.ShapeDtypeStruct((B, 1, 256), F32),
        ],
    )(z0, mean0, var0, row(fp_l0_g), row(fp_l0_beta), fp_l1_W, row(fp_l1_b))

    mean1 = jnp.sum(s1, axis=0) / cnt
    var1 = jnp.sum(q1, axis=0) / cnt - mean1 ** 2

    out = pl.pallas_call(
        _fp3_kernel,
        grid=(B,),
        in_specs=[
            pl.BlockSpec((1, NT, 256), lambda b: (b, 0, 0)),
            pl.BlockSpec((1, 256), lambda b: (0, 0)),
            pl.BlockSpec((1, 256), lambda b: (0, 0)),
            pl.BlockSpec((1, 256), lambda b: (0, 0)),
            pl.BlockSpec((1, 256), lambda b: (0, 0)),
            pl.BlockSpec((32, 256), lambda b: (0, 0)),
            pl.BlockSpec((32, 1), lambda b: (0, 0)),
        ],
        out_specs=pl.BlockSpec((1, 32, NT), lambda b: (b, 0, 0)),
        out_shape=jax.ShapeDtypeStruct((B, 32, NT), F32),
    )(z1, mean1, var1, row(fp_l1_g), row(fp_l1_beta), conv_W,
      conv_b.reshape(32, 1))

    return out
